# Initial kernel scaffold; baseline (speedup 1.0000x reference)
import jax
import jax.numpy as jnp
from jax.experimental import pallas as pl


def kernel(params, atomic_x, atomic_edge_index, atomic_batch, brics_x, brics_edge_index, brics_batch, target):
    raise NotImplementedError("write your pallas kernel here")


# speedup vs baseline: 3.5758x; 3.5758x over previous
import functools

import jax
import jax.numpy as jnp
from jax import lax
from jax.experimental import pallas as pl
from jax.experimental.pallas import tpu as pltpu
from jax.experimental.pallas import tpu_sc as plsc


def _rup(x, m):
    return (x + m - 1) // m * m


_K = 128
_NW = 32


@functools.partial(jax.jit, static_argnames=("npad",))
def _spmm(m_pad, src, dst, npad):
    n, cp = m_pad.shape
    e_pad = src.shape[0]
    epw = e_pad // _NW
    chunks = epw // _K
    rpt = npad // 16
    zeros_hbm = jnp.zeros((rpt, cp), jnp.float32)

    mesh = plsc.VectorSubcoreMesh(core_axis_name="c", subcore_axis_name="s")

    @functools.partial(
        pl.kernel,
        mesh=mesh,
        compiler_params=pltpu.CompilerParams(use_tc_tiling_on_sc=False),
        out_type=jax.ShapeDtypeStruct((2, npad, cp), jnp.float32),
        scratch_types=[
            pltpu.VMEM_SHARED((npad, cp), jnp.float32),
            pltpu.VMEM((_K,), jnp.int32),
            pltpu.VMEM((_K,), jnp.int32),
            pltpu.VMEM((_K, cp), jnp.float32),
            pltpu.SemaphoreType.DMA,
        ],
    )
    def k(m_hbm, src_hbm, dst_hbm, z_hbm, out_hbm, acc, src_v, dst_v, rows_v,
          sem):
        cid = lax.axis_index("c")
        sid = lax.axis_index("s")
        wid = sid * 2 + cid

        r0 = sid * rpt
        pltpu.sync_copy(z_hbm, acc.at[pl.ds(r0, rpt)])
        plsc.subcore_barrier()

        base = wid * epw

        def body(j, carry):
            off = pl.multiple_of(base + j * _K, _K)
            pltpu.sync_copy(src_hbm.at[pl.ds(off, _K)], src_v)
            pltpu.sync_copy(dst_hbm.at[pl.ds(off, _K)], dst_v)
            pltpu.async_copy(m_hbm.at[src_v], rows_v, sem).wait()
            pltpu.sync_copy(rows_v, acc.at[dst_v], add=True)
            return carry

        lax.fori_loop(0, chunks, body, 0)
        plsc.subcore_barrier()

        pltpu.sync_copy(acc.at[pl.ds(r0, rpt)],
                        out_hbm.at[cid, pl.ds(r0, rpt)])

    return k(m_pad, src, dst, zeros_hbm)


def _f32(*shape):
    return jax.ShapeDtypeStruct(shape, jnp.float32)


_VMEM_BIG = pltpu.CompilerParams(vmem_limit_bytes=100 * 1024 * 1024)


_CMAX = 128


def _col_pieces(c):
    pieces, o = [], 0
    while o < c:
        w = min(_CMAX, c - o)
        pieces.append((o, w, _rup(w, 16)))
        o += w
    return pieces


def _make_pre_body(nx, npc):
    def body(*refs):
        xs = refs[:nx]
        wrs = refs[nx:nx + nx * npc]
        wws = refs[nx + nx * npc:nx + nx * npc + nx]
        m_refs = refs[-npc - 1:-1]
        r_ref = refs[-1]
        for pi in range(npc):
            m = jnp.dot(xs[0][...], wrs[0 * npc + pi][...],
                        preferred_element_type=jnp.float32)
            for i in range(1, nx):
                m += jnp.dot(xs[i][...], wrs[i * npc + pi][...],
                             preferred_element_type=jnp.float32)
            m_refs[pi][...] = m
        r = jnp.dot(xs[0][...], wws[0][...], preferred_element_type=jnp.float32)
        for i in range(1, nx):
            r += jnp.dot(xs[i][...], wws[i][...],
                         preferred_element_type=jnp.float32)
        r_ref[...] = r
    return body


def _gcbn_pre(xs, w_rel, w_root, pieces):
    n = xs[0].shape[0]
    c = w_rel.shape[1]
    wrs, wws, o = [], [], 0
    for x in xs:
        wrow = w_rel[o:o + x.shape[1]]
        for (po, pw, pcp) in pieces:
            wrs.append(jnp.pad(wrow[:, po:po + pw], ((0, 0), (0, pcp - pw))))
        wws.append(w_root[o:o + x.shape[1]])
        o += x.shape[1]
    body = _make_pre_body(len(xs), len(pieces))
    rb = 2000 if n % 2000 == 0 else n
    grid = (n // rb,)
    in_specs = ([pl.BlockSpec((rb, x.shape[1]), lambda i: (i, 0)) for x in xs]
                + [pl.BlockSpec(w.shape, lambda i: (0, 0)) for w in wrs]
                + [pl.BlockSpec(w.shape, lambda i: (0, 0)) for w in wws])
    out_specs = ([pl.BlockSpec((rb, pcp), lambda i: (i, 0))
                  for (_, _, pcp) in pieces]
                 + [pl.BlockSpec((rb, c), lambda i: (i, 0))])
    outs = pl.pallas_call(
        body, grid=grid, in_specs=in_specs, out_specs=out_specs,
        out_shape=[_f32(n, pcp) for (_, _, pcp) in pieces] + [_f32(n, c)]
    )(*xs, *wrs, *wws)
    return outs[:-1], outs[-1]


def _gcbn_post_body(n, widths, *refs):
    npc = len(widths)
    agg_refs = refs[:npc]
    r_ref, b_ref, g_ref, be_ref, o_ref = refs[npc:]
    parts = [agg_refs[i][0, :n, :widths[i]] + agg_refs[i][1, :n, :widths[i]]
             for i in range(npc)]
    agg = parts[0] if npc == 1 else jnp.concatenate(parts, axis=1)
    h = agg + r_ref[...] + b_ref[...]
    mean = jnp.mean(h, axis=0, keepdims=True)
    var = jnp.mean((h - mean) ** 2, axis=0, keepdims=True)
    o_ref[...] = jnp.maximum(
        (h - mean) * lax.rsqrt(var + 1e-5) * g_ref[...] + be_ref[...], 0.0)


def _gcbn_post(agg_list, widths, r, b, gamma, beta):
    n, c = r.shape
    return pl.pallas_call(
        functools.partial(_gcbn_post_body, n, widths), out_shape=_f32(n, c),
        compiler_params=_VMEM_BIG)(
        *agg_list, r, b.reshape(1, c), gamma.reshape(1, c), beta.reshape(1, c))


def _pool_cls_body(ng, bid_ref, x_ref, w_ref, b_ref, o_ref):
    n = x_ref.shape[0]
    gids = lax.broadcasted_iota(jnp.int32, (ng, n), 0)
    onehot = (gids == bid_ref[...]).astype(jnp.float32)
    sums = jnp.dot(onehot, x_ref[...], preferred_element_type=jnp.float32)
    cnt = jnp.sum(onehot, axis=1, keepdims=True)
    pooled = sums / jnp.maximum(cnt, 1.0)
    o_ref[...] = jnp.dot(pooled, w_ref[...],
                         preferred_element_type=jnp.float32) + b_ref[...]


def _pool_cls(x, batch_ids, ng, w, b):
    return pl.pallas_call(
        functools.partial(_pool_cls_body, ng),
        out_shape=_f32(ng, w.shape[1]))(
        batch_ids.reshape(1, -1), x, w, b.reshape(1, -1))


_LP = 1002


def _emb_body(vpad, bb, lseq, tok_ref, tbl_ref, o_ref):
    tok = tok_ref[...]
    vids = lax.broadcasted_iota(jnp.int32, (bb * lseq, vpad), 1)
    oh = (tok == vids).astype(jnp.float32)
    emb = jnp.dot(oh, tbl_ref[...], preferred_element_type=jnp.float32)
    d = tbl_ref.shape[1]
    z = jnp.zeros((bb, 1, d), jnp.float32)
    o_ref[...] = jnp.concatenate([z, emb.reshape(bb, lseq, d), z], axis=1)


def _embed(tokens, table):
    b, lseq = tokens.shape
    v, d = table.shape
    vpad = _rup(v, 16)
    tbl = jnp.pad(table, ((0, vpad - v), (0, 0)))
    bb = 16
    return pl.pallas_call(
        functools.partial(_emb_body, vpad, bb, lseq),
        grid=(b // bb,),
        in_specs=[pl.BlockSpec((bb * lseq, 1), lambda i: (i, 0)),
                  pl.BlockSpec((vpad, d), lambda i: (0, 0))],
        out_specs=pl.BlockSpec((bb, _LP, d), lambda i: (i, 0, 0)),
        out_shape=_f32(b, _LP, d))(tokens.reshape(b * lseq, 1), tbl)


def _conv_body(do_max, x_ref, w0_ref, w1_ref, w2_ref, b_ref, o_ref):
    bb, lp, cin = x_ref.shape
    co = w0_ref.shape[1]
    xf = x_ref[...].reshape(bb * lp, cin)
    a = (jnp.dot(xf[:bb * lp - 2], w0_ref[...], preferred_element_type=jnp.float32)
         + jnp.dot(xf[1:bb * lp - 1], w1_ref[...], preferred_element_type=jnp.float32)
         + jnp.dot(xf[2:], w2_ref[...], preferred_element_type=jnp.float32))
    y = jnp.maximum(a + b_ref[...], 0.0)
    yfull = jnp.concatenate(
        [jnp.zeros((1, co), jnp.float32), y, jnp.zeros((1, co), jnp.float32)], axis=0)
    li = lax.rem(lax.broadcasted_iota(jnp.int32, (bb * lp, 1), 0), lp)
    valid = jnp.logical_and(li >= 1, li <= lp - 2).astype(jnp.float32)
    yfull = yfull * valid
    if do_max:
        o_ref[...] = jnp.max(yfull.reshape(bb, lp, co), axis=1)
    else:
        o_ref[...] = yfull.reshape(bb, lp, co)


def _conv1d(x, w, b, do_max):
    bsz, lp, cin = x.shape
    co = w.shape[0]
    wk = jnp.transpose(w, (2, 1, 0))
    bb = 16
    out_shape = _f32(bsz, co) if do_max else _f32(bsz, lp, co)
    out_spec = (pl.BlockSpec((bb, co), lambda i: (i, 0)) if do_max
                else pl.BlockSpec((bb, lp, co), lambda i: (i, 0, 0)))
    return pl.pallas_call(
        functools.partial(_conv_body, do_max),
        grid=(bsz // bb,),
        in_specs=[pl.BlockSpec((bb, lp, cin), lambda i: (i, 0, 0)),
                  pl.BlockSpec((cin, co), lambda i: (0, 0)),
                  pl.BlockSpec((cin, co), lambda i: (0, 0)),
                  pl.BlockSpec((cin, co), lambda i: (0, 0)),
                  pl.BlockSpec((1, co), lambda i: (0, 0))],
        out_specs=out_spec,
        out_shape=out_shape)(x, wk[0], wk[1], wk[2], b.reshape(1, co))


def _mm_body(nx, act, refs):
    xs = refs[:nx]
    ws = refs[nx:2 * nx]
    b_ref, o_ref = refs[2 * nx], refs[2 * nx + 1]
    y = jnp.dot(xs[0][...], ws[0][...], preferred_element_type=jnp.float32)
    for i in range(1, nx):
        y += jnp.dot(xs[i][...], ws[i][...], preferred_element_type=jnp.float32)
    y = y + b_ref[...]
    if act == "relu":
        y = jnp.maximum(y, 0.0)
    elif act == "sigmoid":
        y = jax.nn.sigmoid(y)
    o_ref[...] = y


def _mm(xs, w, b, act="none"):
    n = xs[0].shape[0]
    co = w.shape[1]
    ws, o = [], 0
    for x in xs:
        ws.append(w[o:o + x.shape[1]])
        o += x.shape[1]

    def body(*refs):
        _mm_body(len(xs), act, refs)

    return pl.pallas_call(body, out_shape=_f32(n, co))(
        *xs, *ws, b.reshape(1, co))


def _head_body(a_ref, b2_ref, p_ref, gwa_ref, gwb_ref, gb_ref, fwg_ref,
               fwd_ref, fb_ref, c1p_ref, c1f_ref, b1_ref, w2_ref, bb2_ref,
               w3_ref, b3_ref, o_ref):
    a = a_ref[...]
    b = b2_ref[...]
    prot = p_ref[...]
    g = jax.nn.sigmoid(
        jnp.dot(a, gwa_ref[...], preferred_element_type=jnp.float32)
        + jnp.dot(b, gwb_ref[...], preferred_element_type=jnp.float32)
        + gb_ref[...])
    gated = g * a + (1.0 - g) * b
    fused = (jnp.dot(gated, fwg_ref[...], preferred_element_type=jnp.float32)
             + jnp.dot(a - b, fwd_ref[...], preferred_element_type=jnp.float32)
             + fb_ref[...])
    h = jnp.maximum(
        jnp.dot(prot, c1p_ref[...], preferred_element_type=jnp.float32)
        + jnp.dot(fused, c1f_ref[...], preferred_element_type=jnp.float32)
        + b1_ref[...], 0.0)
    h = jnp.maximum(
        jnp.dot(h, w2_ref[...], preferred_element_type=jnp.float32)
        + bb2_ref[...], 0.0)
    o_ref[...] = (jnp.dot(h, w3_ref[...], preferred_element_type=jnp.float32)
                  + b3_ref[...])


def _head(a, b, prot, params):
    gw, gb = params['gate']['w'], params['gate']['b']
    fw, fb = params['fuse']['w'], params['fuse']['b']
    c1w, c1b = params['cls1']['w'], params['cls1']['b']
    w2, b2 = params['cls2']['w'], params['cls2']['b']
    w3, b3 = params['cls3']['w'], params['cls3']['b']
    ng = a.shape[0]
    return pl.pallas_call(_head_body, out_shape=_f32(ng, 1))(
        a, b, prot,
        gw[:96], gw[96:], gb.reshape(1, -1),
        fw[:96], fw[96:], fb.reshape(1, -1),
        c1w[:96], c1w[96:], c1b.reshape(1, -1),
        w2, b2.reshape(1, -1), w3, b3.reshape(1, -1))


def _gcbn(p, xs, src, dst, npad):
    c = p['w_rel'].shape[1]
    pieces = _col_pieces(c)
    m_list, r = _gcbn_pre(xs, p['w_rel'], p['w_root'], pieces)
    agg_list = [_spmm(m, src, dst, npad) for m in m_list]
    widths = [pw for (_, pw, _) in pieces]
    return _gcbn_post(agg_list, widths, r, p['b_rel'], p['gamma'], p['beta'])


def _encoder(p, x, edge_index, batch_ids, n_graphs, n_blocks):
    n = x.shape[0]
    npad = _rup(n + 1, 128)
    e = edge_index.shape[1]
    e_pad = _rup(e, _NW * _K)
    src = jnp.concatenate(
        [edge_index[0].astype(jnp.int32), jnp.zeros((e_pad - e,), jnp.int32)])
    dst = jnp.concatenate(
        [edge_index[1].astype(jnp.int32),
         jnp.full((e_pad - e,), n, jnp.int32)])

    x = _gcbn(p['conv0'], [x], src, dst, npad)
    for bi in range(n_blocks):
        feats = [x]
        for lp in p['block%d' % (bi + 1)]:
            h = _gcbn(lp['conv1'], feats, src, dst, npad)
            h = _gcbn(lp['conv2'], [h], src, dst, npad)
            feats.append(h)
        x = _gcbn(p['trans%d' % (bi + 1)], feats, src, dst, npad)
    return _pool_cls(x, batch_ids.astype(jnp.int32), n_graphs,
                     p['cls']['w'], p['cls']['b'])


def _protein(p, tokens):
    x = _embed(tokens.astype(jnp.int32), p['table'])
    feats = []
    for convs in p['blocks']:
        h = x
        for ci, c in enumerate(convs):
            last = ci == len(convs) - 1
            h = _conv1d(h, c['w'], c['b'], do_max=last)
        feats.append(h)
    return _mm(feats, p['lin']['w'], p['lin']['b'])


def kernel(params, atomic_x, atomic_edge_index, atomic_batch, brics_x,
           brics_edge_index, brics_batch, target):
    n_graphs = target.shape[0]
    prot = _protein(params['protein'], target)
    a = _encoder(params['atomic'], atomic_x, atomic_edge_index, atomic_batch,
                 n_graphs, 2)
    b = _encoder(params['brics'], brics_x, brics_edge_index, brics_batch,
                 n_graphs, 2)
    return _head(a, b, prot, params)


# speedup vs baseline: 4.6990x; 1.3141x over previous
import functools

import jax
import jax.numpy as jnp
from jax import lax
from jax.experimental import pallas as pl
from jax.experimental.pallas import tpu as pltpu
from jax.experimental.pallas import tpu_sc as plsc


def _rup(x, m):
    return (x + m - 1) // m * m


_K = 128
_NW = 32


@functools.partial(jax.jit, static_argnames=("npad",))
def _spmm(m_pad, src, dst, npad):
    n, cp = m_pad.shape
    e_pad = src.shape[0]
    epw = e_pad // _NW
    chunks = epw // _K
    n_iter = chunks // 2
    rpt = npad // 16
    zeros_hbm = jnp.zeros((rpt, cp), jnp.float32)
    src2 = src.reshape(_NW, epw)
    dst2 = dst.reshape(_NW, chunks, 1, _K)

    mesh = plsc.VectorSubcoreMesh(core_axis_name="c", subcore_axis_name="s")

    @functools.partial(
        pl.kernel,
        mesh=mesh,
        compiler_params=pltpu.CompilerParams(use_tc_tiling_on_sc=False),
        out_type=jax.ShapeDtypeStruct((2, npad, cp), jnp.float32),
        scratch_types=[
            pltpu.VMEM_SHARED((npad, cp), jnp.float32),
            pltpu.VMEM((epw,), jnp.int32),
            pltpu.VMEM((chunks, 1, _K), jnp.int32),
            pltpu.VMEM((_K, cp), jnp.float32),
            pltpu.VMEM((_K, cp), jnp.float32),
            pltpu.SemaphoreType.DMA,
            pltpu.SemaphoreType.DMA,
        ],
    )
    def k(m_hbm, src_hbm, dst_hbm, z_hbm, out_hbm, acc, src_v, dst_v, rows0,
          rows1, gsem0, gsem1):
        cid = lax.axis_index("c")
        sid = lax.axis_index("s")
        wid = sid * 2 + cid

        r0 = sid * rpt
        pltpu.sync_copy(z_hbm, acc.at[pl.ds(r0, rpt)])

        pltpu.sync_copy(src_hbm.at[wid], src_v)
        pltpu.sync_copy(dst_hbm.at[wid], dst_v)
        plsc.subcore_barrier()

        def sidx(j):
            return src_v.at[pl.ds(pl.multiple_of(j * _K, _K), _K)]

        pltpu.async_copy(m_hbm.at[sidx(0)], rows0, gsem0)

        def body(jj, carry):
            j0 = jj * 2
            pltpu.async_copy(m_hbm.at[sidx(j0 + 1)], rows1, gsem1)
            pltpu.make_async_copy(m_hbm.at[sidx(j0)], rows0, gsem0).wait()
            pltpu.sync_copy(rows0, acc.at[dst_v.at[j0, 0]], add=True)

            @pl.when(jj + 1 < n_iter)
            def _():
                pltpu.async_copy(m_hbm.at[sidx(j0 + 2)], rows0, gsem0)

            pltpu.make_async_copy(m_hbm.at[sidx(j0 + 1)], rows1, gsem1).wait()
            pltpu.sync_copy(rows1, acc.at[dst_v.at[j0 + 1, 0]], add=True)
            return carry

        lax.fori_loop(0, n_iter, body, 0)
        plsc.subcore_barrier()

        pltpu.sync_copy(acc.at[pl.ds(r0, rpt)],
                        out_hbm.at[cid, pl.ds(r0, rpt)])

    return k(m_pad, src2, dst2, zeros_hbm)


def _f32(*shape):
    return jax.ShapeDtypeStruct(shape, jnp.float32)


_VMEM_BIG = pltpu.CompilerParams(vmem_limit_bytes=100 * 1024 * 1024)


_CMAX = 128


def _col_pieces(c):
    pieces, o = [], 0
    while o < c:
        w = min(_CMAX, c - o)
        pieces.append((o, w, _rup(w, 16)))
        o += w
    return pieces


def _make_pre_body(nx, npc):
    def body(*refs):
        xs = refs[:nx]
        wrs = refs[nx:nx + nx * npc]
        wws = refs[nx + nx * npc:nx + nx * npc + nx]
        m_refs = refs[-npc - 1:-1]
        r_ref = refs[-1]
        for pi in range(npc):
            m = jnp.dot(xs[0][...], wrs[0 * npc + pi][...],
                        preferred_element_type=jnp.float32)
            for i in range(1, nx):
                m += jnp.dot(xs[i][...], wrs[i * npc + pi][...],
                             preferred_element_type=jnp.float32)
            m_refs[pi][...] = m
        r = jnp.dot(xs[0][...], wws[0][...], preferred_element_type=jnp.float32)
        for i in range(1, nx):
            r += jnp.dot(xs[i][...], wws[i][...],
                         preferred_element_type=jnp.float32)
        r_ref[...] = r
    return body


def _gcbn_pre(xs, w_rel, w_root, pieces):
    n = xs[0].shape[0]
    c = w_rel.shape[1]
    wrs, wws, o = [], [], 0
    for x in xs:
        wrow = w_rel[o:o + x.shape[1]]
        for (po, pw, pcp) in pieces:
            wrs.append(jnp.pad(wrow[:, po:po + pw], ((0, 0), (0, pcp - pw))))
        wws.append(w_root[o:o + x.shape[1]])
        o += x.shape[1]
    body = _make_pre_body(len(xs), len(pieces))
    rb = 2000 if n % 2000 == 0 else n
    grid = (n // rb,)
    in_specs = ([pl.BlockSpec((rb, x.shape[1]), lambda i: (i, 0)) for x in xs]
                + [pl.BlockSpec(w.shape, lambda i: (0, 0)) for w in wrs]
                + [pl.BlockSpec(w.shape, lambda i: (0, 0)) for w in wws])
    out_specs = ([pl.BlockSpec((rb, pcp), lambda i: (i, 0))
                  for (_, _, pcp) in pieces]
                 + [pl.BlockSpec((rb, c), lambda i: (i, 0))])
    outs = pl.pallas_call(
        body, grid=grid, in_specs=in_specs, out_specs=out_specs,
        out_shape=[_f32(n, pcp) for (_, _, pcp) in pieces] + [_f32(n, c)]
    )(*xs, *wrs, *wws)
    return outs[:-1], outs[-1]


def _gcbn_post_body(n, widths, *refs):
    npc = len(widths)
    agg_refs = refs[:npc]
    r_ref, b_ref, g_ref, be_ref, o_ref = refs[npc:]
    parts = [agg_refs[i][0, :n, :widths[i]] + agg_refs[i][1, :n, :widths[i]]
             for i in range(npc)]
    agg = parts[0] if npc == 1 else jnp.concatenate(parts, axis=1)
    h = agg + r_ref[...] + b_ref[...]
    mean = jnp.mean(h, axis=0, keepdims=True)
    var = jnp.mean((h - mean) ** 2, axis=0, keepdims=True)
    o_ref[...] = jnp.maximum(
        (h - mean) * lax.rsqrt(var + 1e-5) * g_ref[...] + be_ref[...], 0.0)


def _gcbn_post(agg_list, widths, r, b, gamma, beta):
    n, c = r.shape
    return pl.pallas_call(
        functools.partial(_gcbn_post_body, n, widths), out_shape=_f32(n, c),
        compiler_params=_VMEM_BIG)(
        *agg_list, r, b.reshape(1, c), gamma.reshape(1, c), beta.reshape(1, c))


def _pool_cls_body(ng, bid_ref, x_ref, w_ref, b_ref, o_ref):
    n = x_ref.shape[0]
    gids = lax.broadcasted_iota(jnp.int32, (ng, n), 0)
    onehot = (gids == bid_ref[...]).astype(jnp.float32)
    sums = jnp.dot(onehot, x_ref[...], preferred_element_type=jnp.float32)
    cnt = jnp.sum(onehot, axis=1, keepdims=True)
    pooled = sums / jnp.maximum(cnt, 1.0)
    o_ref[...] = jnp.dot(pooled, w_ref[...],
                         preferred_element_type=jnp.float32) + b_ref[...]


def _pool_cls(x, batch_ids, ng, w, b):
    return pl.pallas_call(
        functools.partial(_pool_cls_body, ng),
        out_shape=_f32(ng, w.shape[1]))(
        batch_ids.reshape(1, -1), x, w, b.reshape(1, -1))


_LP = 1002


def _emb_body(vpad, bb, lseq, tok_ref, tbl_ref, o_ref):
    tok = tok_ref[...]
    vids = lax.broadcasted_iota(jnp.int32, (bb * lseq, vpad), 1)
    oh = (tok == vids).astype(jnp.float32)
    emb = jnp.dot(oh, tbl_ref[...], preferred_element_type=jnp.float32)
    d = tbl_ref.shape[1]
    z = jnp.zeros((bb, 1, d), jnp.float32)
    o_ref[...] = jnp.concatenate([z, emb.reshape(bb, lseq, d), z], axis=1)


def _embed(tokens, table):
    b, lseq = tokens.shape
    v, d = table.shape
    vpad = _rup(v, 16)
    tbl = jnp.pad(table, ((0, vpad - v), (0, 0)))
    bb = 16
    return pl.pallas_call(
        functools.partial(_emb_body, vpad, bb, lseq),
        grid=(b // bb,),
        in_specs=[pl.BlockSpec((bb * lseq, 1), lambda i: (i, 0)),
                  pl.BlockSpec((vpad, d), lambda i: (0, 0))],
        out_specs=pl.BlockSpec((bb, _LP, d), lambda i: (i, 0, 0)),
        out_shape=_f32(b, _LP, d))(tokens.reshape(b * lseq, 1), tbl)


def _conv_body(do_max, x_ref, w0_ref, w1_ref, w2_ref, b_ref, o_ref):
    bb, lp, cin = x_ref.shape
    co = w0_ref.shape[1]
    xf = x_ref[...].reshape(bb * lp, cin)
    a = (jnp.dot(xf[:bb * lp - 2], w0_ref[...], preferred_element_type=jnp.float32)
         + jnp.dot(xf[1:bb * lp - 1], w1_ref[...], preferred_element_type=jnp.float32)
         + jnp.dot(xf[2:], w2_ref[...], preferred_element_type=jnp.float32))
    y = jnp.maximum(a + b_ref[...], 0.0)
    yfull = jnp.concatenate(
        [jnp.zeros((1, co), jnp.float32), y, jnp.zeros((1, co), jnp.float32)], axis=0)
    li = lax.rem(lax.broadcasted_iota(jnp.int32, (bb * lp, 1), 0), lp)
    valid = jnp.logical_and(li >= 1, li <= lp - 2).astype(jnp.float32)
    yfull = yfull * valid
    if do_max:
        o_ref[...] = jnp.max(yfull.reshape(bb, lp, co), axis=1)
    else:
        o_ref[...] = yfull.reshape(bb, lp, co)


def _conv1d(x, w, b, do_max):
    bsz, lp, cin = x.shape
    co = w.shape[0]
    wk = jnp.transpose(w, (2, 1, 0))
    bb = 16
    out_shape = _f32(bsz, co) if do_max else _f32(bsz, lp, co)
    out_spec = (pl.BlockSpec((bb, co), lambda i: (i, 0)) if do_max
                else pl.BlockSpec((bb, lp, co), lambda i: (i, 0, 0)))
    return pl.pallas_call(
        functools.partial(_conv_body, do_max),
        grid=(bsz // bb,),
        in_specs=[pl.BlockSpec((bb, lp, cin), lambda i: (i, 0, 0)),
                  pl.BlockSpec((cin, co), lambda i: (0, 0)),
                  pl.BlockSpec((cin, co), lambda i: (0, 0)),
                  pl.BlockSpec((cin, co), lambda i: (0, 0)),
                  pl.BlockSpec((1, co), lambda i: (0, 0))],
        out_specs=out_spec,
        out_shape=out_shape)(x, wk[0], wk[1], wk[2], b.reshape(1, co))


def _mm_body(nx, act, refs):
    xs = refs[:nx]
    ws = refs[nx:2 * nx]
    b_ref, o_ref = refs[2 * nx], refs[2 * nx + 1]
    y = jnp.dot(xs[0][...], ws[0][...], preferred_element_type=jnp.float32)
    for i in range(1, nx):
        y += jnp.dot(xs[i][...], ws[i][...], preferred_element_type=jnp.float32)
    y = y + b_ref[...]
    if act == "relu":
        y = jnp.maximum(y, 0.0)
    elif act == "sigmoid":
        y = jax.nn.sigmoid(y)
    o_ref[...] = y


def _mm(xs, w, b, act="none"):
    n = xs[0].shape[0]
    co = w.shape[1]
    ws, o = [], 0
    for x in xs:
        ws.append(w[o:o + x.shape[1]])
        o += x.shape[1]

    def body(*refs):
        _mm_body(len(xs), act, refs)

    return pl.pallas_call(body, out_shape=_f32(n, co))(
        *xs, *ws, b.reshape(1, co))


def _head_body(a_ref, b2_ref, p_ref, gwa_ref, gwb_ref, gb_ref, fwg_ref,
               fwd_ref, fb_ref, c1p_ref, c1f_ref, b1_ref, w2_ref, bb2_ref,
               w3_ref, b3_ref, o_ref):
    a = a_ref[...]
    b = b2_ref[...]
    prot = p_ref[...]
    g = jax.nn.sigmoid(
        jnp.dot(a, gwa_ref[...], preferred_element_type=jnp.float32)
        + jnp.dot(b, gwb_ref[...], preferred_element_type=jnp.float32)
        + gb_ref[...])
    gated = g * a + (1.0 - g) * b
    fused = (jnp.dot(gated, fwg_ref[...], preferred_element_type=jnp.float32)
             + jnp.dot(a - b, fwd_ref[...], preferred_element_type=jnp.float32)
             + fb_ref[...])
    h = jnp.maximum(
        jnp.dot(prot, c1p_ref[...], preferred_element_type=jnp.float32)
        + jnp.dot(fused, c1f_ref[...], preferred_element_type=jnp.float32)
        + b1_ref[...], 0.0)
    h = jnp.maximum(
        jnp.dot(h, w2_ref[...], preferred_element_type=jnp.float32)
        + bb2_ref[...], 0.0)
    o_ref[...] = (jnp.dot(h, w3_ref[...], preferred_element_type=jnp.float32)
                  + b3_ref[...])


def _head(a, b, prot, params):
    gw, gb = params['gate']['w'], params['gate']['b']
    fw, fb = params['fuse']['w'], params['fuse']['b']
    c1w, c1b = params['cls1']['w'], params['cls1']['b']
    w2, b2 = params['cls2']['w'], params['cls2']['b']
    w3, b3 = params['cls3']['w'], params['cls3']['b']
    ng = a.shape[0]
    return pl.pallas_call(_head_body, out_shape=_f32(ng, 1))(
        a, b, prot,
        gw[:96], gw[96:], gb.reshape(1, -1),
        fw[:96], fw[96:], fb.reshape(1, -1),
        c1w[:96], c1w[96:], c1b.reshape(1, -1),
        w2, b2.reshape(1, -1), w3, b3.reshape(1, -1))


def _gcbn(p, xs, src, dst, npad):
    c = p['w_rel'].shape[1]
    pieces = _col_pieces(c)
    m_list, r = _gcbn_pre(xs, p['w_rel'], p['w_root'], pieces)
    agg_list = [_spmm(m, src, dst, npad) for m in m_list]
    widths = [pw for (_, pw, _) in pieces]
    return _gcbn_post(agg_list, widths, r, p['b_rel'], p['gamma'], p['beta'])


def _encoder(p, x, edge_index, batch_ids, n_graphs, n_blocks):
    n = x.shape[0]
    npad = _rup(n + 1, 128)
    e = edge_index.shape[1]
    e_pad = _rup(e, _NW * _K * 2)
    src = jnp.concatenate(
        [edge_index[0].astype(jnp.int32), jnp.zeros((e_pad - e,), jnp.int32)])
    dst = jnp.concatenate(
        [edge_index[1].astype(jnp.int32),
         jnp.full((e_pad - e,), n, jnp.int32)])

    x = _gcbn(p['conv0'], [x], src, dst, npad)
    for bi in range(n_blocks):
        feats = [x]
        for lp in p['block%d' % (bi + 1)]:
            h = _gcbn(lp['conv1'], feats, src, dst, npad)
            h = _gcbn(lp['conv2'], [h], src, dst, npad)
            feats.append(h)
        x = _gcbn(p['trans%d' % (bi + 1)], feats, src, dst, npad)
    return _pool_cls(x, batch_ids.astype(jnp.int32), n_graphs,
                     p['cls']['w'], p['cls']['b'])


def _protein(p, tokens):
    x = _embed(tokens.astype(jnp.int32), p['table'])
    feats = []
    for convs in p['blocks']:
        h = x
        for ci, c in enumerate(convs):
            last = ci == len(convs) - 1
            h = _conv1d(h, c['w'], c['b'], do_max=last)
        feats.append(h)
    return _mm(feats, p['lin']['w'], p['lin']['b'])


def kernel(params, atomic_x, atomic_edge_index, atomic_batch, brics_x,
           brics_edge_index, brics_batch, target):
    n_graphs = target.shape[0]
    prot = _protein(params['protein'], target)
    a = _encoder(params['atomic'], atomic_x, atomic_edge_index, atomic_batch,
                 n_graphs, 2)
    b = _encoder(params['brics'], brics_x, brics_edge_index, brics_batch,
                 n_graphs, 2)
    return _head(a, b, prot, params)
